# speedup vs baseline: 129.7079x; 129.7079x over previous
import jax
import jax.numpy as jnp
from jax.experimental import pallas as pl
from jax.experimental.pallas import tpu as pltpu


def _body(pred_ref, anc_ref, tgt_ref, out_ref, cost_ref):
    G, Q = cost_ref.shape
    tgt = tgt_ref[0]
    tcx = (tgt[:, 0:1] + tgt[:, 2:3]) / 2
    tcy = (tgt[:, 1:2] + tgt[:, 3:4]) / 2
    tw = tgt[:, 2:3] - tgt[:, 0:1]
    th = tgt[:, 3:4] - tgt[:, 1:2]
    lane_iota = jax.lax.broadcasted_iota(jnp.int32, (G, Q), 1)

    def topk_into(bx_ref, col0):
        x0 = bx_ref[0, 0:1, :]
        y0 = bx_ref[0, 1:2, :]
        x1 = bx_ref[0, 2:3, :]
        y1 = bx_ref[0, 3:4, :]
        cx = (x0 + x1) / 2
        cy = (y0 + y1) / 2
        w = x1 - x0
        h = y1 - y0
        s = jnp.abs(cx - tcx)
        s = s + jnp.abs(cy - tcy)
        s = s + jnp.abs(w - tw)
        s = s + jnp.abs(h - th)
        cost_ref[...] = s
        for r in range(4):
            c = cost_ref[...]
            mval = jnp.min(c, axis=1, keepdims=True)
            idx = jnp.min(
                jnp.where(c == mval, lane_iota, jnp.int32(Q)),
                axis=1, keepdims=True)
            out_ref[0, :, col0 + r:col0 + r + 1] = idx
            if r < 3:
                cost_ref[...] = jnp.where(lane_iota == idx, jnp.inf, c)

    topk_into(pred_ref, 0)
    topk_into(anc_ref, 4)


def kernel(pred_boxes, anchors, tgt_boxes):
    bs, Q, _ = pred_boxes.shape
    G = tgt_boxes.shape[1]
    K = 4
    pred_t = jnp.transpose(pred_boxes, (0, 2, 1))
    anc_t = jnp.transpose(anchors, (0, 2, 1))

    out = pl.pallas_call(
        _body,
        grid=(bs,),
        in_specs=[
            pl.BlockSpec((1, 4, Q), lambda b: (b, 0, 0)),
            pl.BlockSpec((1, 4, Q), lambda b: (b, 0, 0)),
            pl.BlockSpec((1, G, 4), lambda b: (b, 0, 0)),
        ],
        out_specs=pl.BlockSpec((1, G, 2 * K), lambda b: (b, 0, 0)),
        out_shape=jax.ShapeDtypeStruct((bs, G, 2 * K), jnp.int32),
        scratch_shapes=[pltpu.VMEM((G, Q), jnp.float32)],
    )(pred_t, anc_t, tgt_boxes)

    idx_i = out.reshape(bs, G * 2 * K).astype(jnp.int64)
    jrow = jnp.concatenate([jnp.arange(K), jnp.arange(K)])
    idx_j = jnp.tile(jrow, (bs, G)).astype(jnp.int64)
    return (idx_i, idx_j)
